# Initial kernel scaffold; baseline (speedup 1.0000x reference)
import jax
import jax.numpy as jnp
from jax.experimental import pallas as pl


def kernel(x_nchw, w1, b1, w2, b2, w3, b3, wl1, bl1, wl2, bl2):
    raise NotImplementedError("write your pallas kernel here")


# speedup vs baseline: 1.0259x; 1.0259x over previous
import numpy as np

import jax
import jax.numpy as jnp
from jax.experimental import pallas as pl
from jax.experimental.pallas import tpu as pltpu


def _build_w1_gather():
    gmap = np.full((4, 144), 27, dtype=np.int32)
    for a in range(2):
        for b in range(2):
            for dh in range(3):
                for dw in range(3):
                    for c in range(3):
                        r = a + dh - 1
                        s = b + dw - 1
                        di, p = r // 2, r % 2
                        dj, q = s // 2, s % 2
                        row = ((di + 1) * 3 + (dj + 1)) * 16 + (p * 6 + q * 3 + c)
                        gmap[a * 2 + b, row] = c * 9 + dh * 3 + dw
    return gmap


_W1_GMAP = _build_w1_gather()


def _pack_w1(w1):
    wz = jnp.concatenate([w1, jnp.zeros((1, 32), w1.dtype)], axis=0)
    blocks = [wz[_W1_GMAP[ab]] for ab in range(4)]
    return jnp.concatenate(blocks, axis=1)


def _im2col(src_ref, dst_ref, rows, width, chans):
    slabs = [
        src_ref[dh:dh + rows, 7 + dw:7 + dw + width, :].reshape(rows * width, chans)
        for dh in range(3) for dw in range(3)
    ]
    per = 128 // chans
    for g in range(0, 9, per):
        hi = min(9, g + per)
        if hi - g > 1:
            dst_ref[:, g * chans:hi * chans] = jnp.concatenate(slabs[g:hi], axis=1)
        else:
            dst_ref[:, g * chans:hi * chans] = slabs[g]


def _pool2(y, rows, width, n):
    y = y.reshape(rows * width // 2, 2, n)
    y = jnp.maximum(y[:, 0, :], y[:, 1, :])
    y = y.reshape(rows // 2, 2, width // 2, n)
    return jnp.maximum(y[:, 0], y[:, 1])


def _conv_kernel(x_ref, w1_ref, b1_ref, w2_ref, b2_ref, w3_ref, b3_ref,
                 out_ref, a0_ref, a1_ref, a2_ref, x1c_ref, x2c_ref, x3c_ref):
    zero = lambda shape: jnp.zeros(shape, jnp.bfloat16)

    a0_ref[0:1, :, :] = zero((1, 80, 16))
    a0_ref[65:66, :, :] = zero((1, 80, 16))
    a0_ref[:, 0:8, :] = zero((66, 8, 16))
    a0_ref[:, 72:80, :] = zero((66, 8, 16))
    a1_ref[0:1, :, :] = zero((1, 80, 32))
    a1_ref[65:66, :, :] = zero((1, 80, 32))
    a1_ref[:, 0:8, :] = zero((66, 8, 32))
    a1_ref[:, 72:80, :] = zero((66, 8, 32))
    a2_ref[0:1, :, :] = zero((1, 48, 64))
    a2_ref[33:34, :, :] = zero((1, 48, 64))
    a2_ref[:, 0:8, :] = zero((34, 8, 64))
    a2_ref[:, 40:48, :] = zero((34, 8, 64))

    a0_ref[1:65, 8:72, :] = x_ref[0]
    _im2col(a0_ref, x1c_ref, 64, 64, 16)
    y = jnp.dot(x1c_ref[...], w1_ref[...], preferred_element_type=jnp.float32)
    y = y.reshape(4096, 4, 32)
    y = jnp.maximum(jnp.maximum(y[:, 0], y[:, 1]),
                    jnp.maximum(y[:, 2], y[:, 3]))
    y = jnp.maximum(y + b1_ref[...], 0.0)
    a1_ref[1:65, 8:72, :] = y.reshape(64, 64, 32).astype(jnp.bfloat16)

    _im2col(a1_ref, x2c_ref, 64, 64, 32)
    y = jnp.dot(x2c_ref[...], w2_ref[...], preferred_element_type=jnp.float32)
    y = _pool2(y, 64, 64, 64)
    y = jnp.maximum(y + b2_ref[...], 0.0)
    a2_ref[1:33, 8:40, :] = y.astype(jnp.bfloat16)

    _im2col(a2_ref, x3c_ref, 32, 32, 64)
    y = jnp.dot(x3c_ref[...], w3_ref[...], preferred_element_type=jnp.float32)
    y = _pool2(y, 32, 32, 128)
    y = jnp.maximum(y + b3_ref[...], 0.0)
    out_ref[0] = y.astype(out_ref.dtype)


def _conv_stack(x16, w1s, b1, w2, b2, w3, b3):
    B = x16.shape[0]
    return pl.pallas_call(
        _conv_kernel,
        out_shape=jax.ShapeDtypeStruct((B, 16, 16, 128), jnp.bfloat16),
        grid_spec=pltpu.PrefetchScalarGridSpec(
            num_scalar_prefetch=0,
            grid=(B,),
            in_specs=[
                pl.BlockSpec((1, 64, 64, 16), lambda b: (b, 0, 0, 0)),
                pl.BlockSpec((144, 128), lambda b: (0, 0)),
                pl.BlockSpec((1, 32), lambda b: (0, 0)),
                pl.BlockSpec((288, 64), lambda b: (0, 0)),
                pl.BlockSpec((1, 64), lambda b: (0, 0)),
                pl.BlockSpec((576, 128), lambda b: (0, 0)),
                pl.BlockSpec((1, 128), lambda b: (0, 0)),
            ],
            out_specs=pl.BlockSpec((1, 16, 16, 128), lambda b: (b, 0, 0, 0)),
            scratch_shapes=[
                pltpu.VMEM((66, 80, 16), jnp.bfloat16),
                pltpu.VMEM((66, 80, 32), jnp.bfloat16),
                pltpu.VMEM((34, 48, 64), jnp.bfloat16),
                pltpu.VMEM((4096, 144), jnp.bfloat16),
                pltpu.VMEM((4096, 288), jnp.bfloat16),
                pltpu.VMEM((1024, 576), jnp.bfloat16),
            ],
        ),
        compiler_params=pltpu.CompilerParams(
            dimension_semantics=("parallel",),
            vmem_limit_bytes=64 * 1024 * 1024,
        ),
    )(x16, w1s, b1, w2, b2, w3, b3)


def _head_kernel(x_ref, w1_ref, b1_ref, w2_ref, b2_ref, out_ref, acc_ref):
    k = pl.program_id(1)

    @pl.when(k == 0)
    def _():
        acc_ref[...] = jnp.zeros_like(acc_ref)

    acc_ref[...] += jnp.dot(x_ref[...], w1_ref[...],
                            preferred_element_type=jnp.float32)

    @pl.when(k == pl.num_programs(1) - 1)
    def _():
        h = acc_ref[...] + b1_ref[...]
        out_ref[...] = (jnp.dot(h, w2_ref[...],
                                preferred_element_type=jnp.float32)
                        + b2_ref[...])


def _head(x_flat, wl1, bl1, wl2, bl2, *, bm=128, tk=4096):
    B, K = x_flat.shape
    H = wl1.shape[1]
    O = wl2.shape[1]
    nb, nk = B // bm, K // tk
    return pl.pallas_call(
        _head_kernel,
        out_shape=jax.ShapeDtypeStruct((B, O), jnp.float32),
        grid_spec=pltpu.PrefetchScalarGridSpec(
            num_scalar_prefetch=0,
            grid=(nb, nk),
            in_specs=[
                pl.BlockSpec((bm, tk), lambda i, k: (i, k)),
                pl.BlockSpec((tk, H), lambda i, k: (k, 0)),
                pl.BlockSpec((1, H), lambda i, k: (0, 0)),
                pl.BlockSpec((H, O), lambda i, k: (0, 0)),
                pl.BlockSpec((1, O), lambda i, k: (0, 0)),
            ],
            out_specs=pl.BlockSpec((bm, O), lambda i, k: (i, 0)),
            scratch_shapes=[pltpu.VMEM((bm, H), jnp.float32)],
        ),
        compiler_params=pltpu.CompilerParams(
            dimension_semantics=("parallel", "arbitrary"),
            vmem_limit_bytes=64 * 1024 * 1024,
        ),
    )(x_flat, wl1, bl1, wl2, bl2)


def kernel(x_nchw, w1, b1, w2, b2, w3, b3, wl1, bl1, wl2, bl2):
    B = x_nchw.shape[0]
    x16 = (x_nchw.reshape(B, 3, 64, 2, 64, 2)
                 .transpose(0, 2, 4, 3, 5, 1)
                 .reshape(B, 64, 64, 12))
    x16 = jnp.pad(x16, ((0, 0), (0, 0), (0, 0), (0, 4))).astype(jnp.bfloat16)
    w1s = _pack_w1(w1)
    act = _conv_stack(x16, w1s, b1, w2, b2, w3, b3)
    return _head(act.reshape(B, 16 * 16 * 128), wl1, bl1, wl2, bl2)


# speedup vs baseline: 2.5479x; 2.4837x over previous
import numpy as np

import jax
import jax.numpy as jnp
from jax.experimental import pallas as pl
from jax.experimental.pallas import tpu as pltpu


def _build_w1_gather():
    gmap = np.full((16, 576), 27, dtype=np.int32)
    for a in range(2):
        for b in range(2):
            for u in range(2):
                for v in range(2):
                    g = (a * 2 + b) * 4 + (u * 2 + v)
                    for dh in range(3):
                        for dw in range(3):
                            for c in range(3):
                                r = 2 * u + a + dh - 1
                                s = 2 * v + b + dw - 1
                                di, rr = r // 4, r % 4
                                dj, ss = s // 4, s % 4
                                row = (((di + 1) * 3 + (dj + 1)) * 64
                                       + (rr * 4 + ss) * 3 + c)
                                gmap[g, row] = c * 9 + dh * 3 + dw
    return gmap


def _build_w2_gather():
    gmap = np.full((4, 1152), 288, dtype=np.int32)
    for a in range(2):
        for b in range(2):
            for dh in range(3):
                for dw in range(3):
                    for c1 in range(32):
                        r = a + dh - 1
                        s = b + dw - 1
                        di, p = r // 2, r % 2
                        dj, q = s // 2, s % 2
                        row = (((di + 1) * 3 + (dj + 1)) * 128
                               + (p * 2 + q) * 32 + c1)
                        gmap[a * 2 + b, row] = dh * 96 + dw * 32 + c1
    return gmap


_W1_GMAP = _build_w1_gather()
_W2_GMAP = _build_w2_gather()


def _pack_w1(w1):
    wz = jnp.concatenate([w1, jnp.zeros((1, 32), w1.dtype)], axis=0)
    return jnp.concatenate([wz[_W1_GMAP[g]] for g in range(16)], axis=1)


def _pack_w2(w2):
    wz = jnp.concatenate([w2, jnp.zeros((1, 64), w2.dtype)], axis=0)
    return jnp.concatenate([wz[_W2_GMAP[g]] for g in range(4)], axis=1)


def _im2col(src_ref, dst_ref, rows, width, chans):
    slabs = [
        src_ref[dh:dh + rows, 7 + dw:7 + dw + width, :].reshape(rows * width, chans)
        for dh in range(3) for dw in range(3)
    ]
    per = max(1, 128 // chans)
    for g in range(0, 9, per):
        hi = min(9, g + per)
        if hi - g > 1:
            dst_ref[:, g * chans:hi * chans] = jnp.concatenate(slabs[g:hi], axis=1)
        else:
            dst_ref[:, g * chans:hi * chans] = slabs[g]


def _quad_max(y, n):
    y = y.reshape(y.shape[0], 4, n)
    return jnp.maximum(jnp.maximum(y[:, 0], y[:, 1]),
                       jnp.maximum(y[:, 2], y[:, 3]))


def _conv_kernel(x_ref, w1_ref, b1_ref, w2_ref, b2_ref, w3_ref, b3_ref,
                 out_ref, a0_ref, a1_ref, a2_ref, x1c_ref, x2c_ref, x3c_ref):
    zero = lambda shape: jnp.zeros(shape, jnp.bfloat16)

    a0_ref[0:1, :, :] = zero((1, 48, 64))
    a0_ref[33:34, :, :] = zero((1, 48, 64))
    a0_ref[:, 0:8, :] = zero((34, 8, 64))
    a0_ref[:, 40:48, :] = zero((34, 8, 64))
    a1_ref[0:1, :, :] = zero((1, 48, 128))
    a1_ref[33:34, :, :] = zero((1, 48, 128))
    a1_ref[:, 0:8, :] = zero((34, 8, 128))
    a1_ref[:, 40:48, :] = zero((34, 8, 128))
    a2_ref[0:1, :, :] = zero((1, 48, 64))
    a2_ref[33:34, :, :] = zero((1, 48, 64))
    a2_ref[:, 0:8, :] = zero((34, 8, 64))
    a2_ref[:, 40:48, :] = zero((34, 8, 64))

    a0_ref[1:33, 8:40, :] = x_ref[0]
    _im2col(a0_ref, x1c_ref, 32, 32, 64)
    y = jnp.dot(x1c_ref[...], w1_ref[...], preferred_element_type=jnp.float32)
    y = _quad_max(y, 128)
    y = jnp.maximum(y + b1_ref[...], 0.0)
    a1_ref[1:33, 8:40, :] = y.reshape(32, 32, 128).astype(jnp.bfloat16)

    _im2col(a1_ref, x2c_ref, 32, 32, 128)
    y = jnp.dot(x2c_ref[...], w2_ref[...], preferred_element_type=jnp.float32)
    y = _quad_max(y, 64)
    y = jnp.maximum(y + b2_ref[...], 0.0)
    a2_ref[1:33, 8:40, :] = y.reshape(32, 32, 64).astype(jnp.bfloat16)

    _im2col(a2_ref, x3c_ref, 32, 32, 64)
    y = jnp.dot(x3c_ref[...], w3_ref[...], preferred_element_type=jnp.float32)
    y = y.reshape(512, 2, 128)
    y = jnp.maximum(y[:, 0, :], y[:, 1, :])
    y = y.reshape(16, 2, 16, 128)
    y = jnp.maximum(y[:, 0], y[:, 1])
    y = jnp.maximum(y + b3_ref[...], 0.0)
    out_ref[0] = y.astype(out_ref.dtype)


def _conv_stack(x4, w1s, b1s, w2s, b2, w3, b3):
    B = x4.shape[0]
    return pl.pallas_call(
        _conv_kernel,
        out_shape=jax.ShapeDtypeStruct((B, 16, 16, 128), jnp.bfloat16),
        grid_spec=pltpu.PrefetchScalarGridSpec(
            num_scalar_prefetch=0,
            grid=(B,),
            in_specs=[
                pl.BlockSpec((1, 32, 32, 64), lambda b: (b, 0, 0, 0)),
                pl.BlockSpec((576, 512), lambda b: (0, 0)),
                pl.BlockSpec((1, 128), lambda b: (0, 0)),
                pl.BlockSpec((1152, 256), lambda b: (0, 0)),
                pl.BlockSpec((1, 64), lambda b: (0, 0)),
                pl.BlockSpec((576, 128), lambda b: (0, 0)),
                pl.BlockSpec((1, 128), lambda b: (0, 0)),
            ],
            out_specs=pl.BlockSpec((1, 16, 16, 128), lambda b: (b, 0, 0, 0)),
            scratch_shapes=[
                pltpu.VMEM((34, 48, 64), jnp.bfloat16),
                pltpu.VMEM((34, 48, 128), jnp.bfloat16),
                pltpu.VMEM((34, 48, 64), jnp.bfloat16),
                pltpu.VMEM((1024, 576), jnp.bfloat16),
                pltpu.VMEM((1024, 1152), jnp.bfloat16),
                pltpu.VMEM((1024, 576), jnp.bfloat16),
            ],
        ),
        compiler_params=pltpu.CompilerParams(
            dimension_semantics=("parallel",),
            vmem_limit_bytes=64 * 1024 * 1024,
        ),
    )(x4, w1s, b1s, w2s, b2, w3, b3)


def _head_kernel(x_ref, w1_ref, b1_ref, w2_ref, b2_ref, out_ref, acc_ref):
    k = pl.program_id(1)

    @pl.when(k == 0)
    def _():
        acc_ref[...] = jnp.zeros_like(acc_ref)

    acc_ref[...] += jnp.dot(x_ref[...], w1_ref[...],
                            preferred_element_type=jnp.float32)

    @pl.when(k == pl.num_programs(1) - 1)
    def _():
        h = acc_ref[...] + b1_ref[...]
        out_ref[...] = (jnp.dot(h, w2_ref[...],
                                preferred_element_type=jnp.float32)
                        + b2_ref[...])


def _head(x_flat, wl1, bl1, wl2, bl2, *, bm=128, tk=4096):
    B, K = x_flat.shape
    H = wl1.shape[1]
    O = wl2.shape[1]
    nb, nk = B // bm, K // tk
    return pl.pallas_call(
        _head_kernel,
        out_shape=jax.ShapeDtypeStruct((B, O), jnp.float32),
        grid_spec=pltpu.PrefetchScalarGridSpec(
            num_scalar_prefetch=0,
            grid=(nb, nk),
            in_specs=[
                pl.BlockSpec((bm, tk), lambda i, k: (i, k)),
                pl.BlockSpec((tk, H), lambda i, k: (k, 0)),
                pl.BlockSpec((1, H), lambda i, k: (0, 0)),
                pl.BlockSpec((H, O), lambda i, k: (0, 0)),
                pl.BlockSpec((1, O), lambda i, k: (0, 0)),
            ],
            out_specs=pl.BlockSpec((bm, O), lambda i, k: (i, 0)),
            scratch_shapes=[pltpu.VMEM((bm, H), jnp.float32)],
        ),
        compiler_params=pltpu.CompilerParams(
            dimension_semantics=("parallel", "arbitrary"),
            vmem_limit_bytes=64 * 1024 * 1024,
        ),
    )(x_flat, wl1, bl1, wl2, bl2)


def kernel(x_nchw, w1, b1, w2, b2, w3, b3, wl1, bl1, wl2, bl2):
    B = x_nchw.shape[0]
    x4 = (x_nchw.reshape(B, 3, 32, 4, 32, 4)
                .transpose(0, 2, 4, 3, 5, 1)
                .reshape(B, 32, 32, 48))
    x4 = jnp.pad(x4, ((0, 0), (0, 0), (0, 0), (0, 16))).astype(jnp.bfloat16)
    w1s = _pack_w1(w1)
    w2s = _pack_w2(w2)
    b1s = jnp.tile(b1, (1, 4))
    act = _conv_stack(x4, w1s, b1s, w2s, b2, w3, b3)
    return _head(act.reshape(B, 16 * 16 * 128), wl1, bl1, wl2, bl2)
